# baseline (speedup 1.0000x reference)
import functools

import jax
import jax.numpy as jnp
from jax import lax
from jax.experimental import pallas as pl
from jax.experimental.pallas import tpu as pltpu
from jax.experimental.pallas import tpu_sc as plsc

D = 64
SCALE = 8.0
NC = 2
NS = 16
NW = NC * NS
CH = 128
NBUF = 4


@functools.lru_cache(maxsize=None)
def _make(B: int):
    assert B % (NW * CH) == 0
    BPW = B // NW
    G = BPW // CH
    mesh = plsc.VectorSubcoreMesh(core_axis_name="c", subcore_axis_name="s")

    @functools.partial(
        pl.kernel,
        mesh=mesh,
        out_type=jax.ShapeDtypeStruct((B, D), jnp.float32),
        compiler_params=pltpu.CompilerParams(use_tc_tiling_on_sc=False),
        scratch_types=[
            pltpu.VMEM((G, CH), jnp.int32),
            *[pltpu.VMEM((CH, D), jnp.float32) for _ in range(NBUF)],
            *[pltpu.SemaphoreType.DMA for _ in range(2 * NBUF)],
        ],
    )
    def emb(x_hbm, lut_hbm, out_hbm, idx_v, r0, r1, r2, r3,
            g0, g1, g2, g3, s0, s1, s2, s3):
        bufs = (r0, r1, r2, r3)
        gsem = (g0, g1, g2, g3)
        ssem = (s0, s1, s2, s3)
        wid = lax.axis_index("s") * NC + lax.axis_index("c")
        base = wid * BPW

        pltpu.sync_copy(x_hbm.at[wid], idx_v)

        def start_gather(g, b):
            pltpu.async_copy(lut_hbm.at[idx_v.at[g]], bufs[b], gsem[b])

        def wait_gather(g, b):
            pltpu.make_async_copy(lut_hbm.at[idx_v.at[g]], bufs[b],
                                  gsem[b]).wait()

        def start_scatter(g, b):
            pltpu.async_copy(bufs[b], out_hbm.at[pl.ds(base + g * CH, CH)],
                             ssem[b])

        def wait_scatter(g, b):
            pltpu.make_async_copy(bufs[b],
                                  out_hbm.at[pl.ds(base + g * CH, CH)],
                                  ssem[b]).wait()

        def scale(b):
            buf = bufs[b]

            def row(r, carry):
                for c in range(D // 16):
                    buf[r, pl.ds(c * 16, 16)] = (
                        buf[r, pl.ds(c * 16, 16)] * SCALE)
                return carry

            lax.fori_loop(0, CH, row, 0)

        start_gather(0, 0)
        start_gather(1, 1)

        def body(i, carry):
            for b in range(NBUF):
                g = i * NBUF + b
                bn = (b + 2) % NBUF
                pl.when(g >= 2)(lambda: wait_scatter(g - 2, bn))
                pl.when(g + 2 < G)(lambda: start_gather(g + 2, bn))
                wait_gather(g, b)
                scale(b)
                start_scatter(g, b)
            return carry

        lax.fori_loop(0, G // NBUF, body, 0)

        wait_scatter(G - 2, (G - 2) % NBUF)
        wait_scatter(G - 1, (G - 1) % NBUF)

    return emb


def kernel(x, lut):
    orig_shape = x.shape
    B = x.size
    xf = x.reshape(NW, B // (NW * CH), CH).astype(jnp.int32)
    out = _make(B)(xf, lut)
    return out.reshape(*orig_shape, D)
